# baseline (speedup 1.0000x reference)
import functools

import jax
import jax.numpy as jnp
from jax import lax
from jax.experimental import pallas as pl
from jax.experimental.pallas import tpu as pltpu
from jax.experimental.pallas import tpu_sc as plsc

BATCH = 16384
FACTOR = 32

NUM_CORES = 2
NUM_SUBCORES = 16
NUM_WORKERS = NUM_CORES * NUM_SUBCORES
BPW = BATCH // NUM_WORKERS
CHUNK = 128
NCH = BPW // CHUNK


def _sc_gather_body(uidx_hbm, iidx_hbm, utab_hbm, itab_hbm,
                    uout_hbm, iout_hbm,
                    idx_v, urows_v, irows_v, sem):
  wid = lax.axis_index("s") * NUM_CORES + lax.axis_index("c")
  base = wid * BPW
  for j in range(NCH):
    pltpu.sync_copy(uidx_hbm.at[pl.ds(base + j * CHUNK, CHUNK)],
                    idx_v.at[j])
    pltpu.sync_copy(iidx_hbm.at[pl.ds(base + j * CHUNK, CHUNK)],
                    idx_v.at[NCH + j])
  copies = []
  for j in range(NCH):
    copies.append(pltpu.async_copy(
        utab_hbm.at[idx_v.at[j]],
        urows_v.at[pl.ds(j * CHUNK, CHUNK)], sem))
    copies.append(pltpu.async_copy(
        itab_hbm.at[idx_v.at[NCH + j]],
        irows_v.at[pl.ds(j * CHUNK, CHUNK)], sem))
  for c in copies:
    c.wait()
  pltpu.sync_copy(urows_v, uout_hbm.at[pl.ds(base, BPW)])
  pltpu.sync_copy(irows_v, iout_hbm.at[pl.ds(base, BPW)])


_sc_gather = functools.partial(
    pl.kernel,
    out_type=(
        jax.ShapeDtypeStruct((BATCH, FACTOR), jnp.float32),
        jax.ShapeDtypeStruct((BATCH, FACTOR), jnp.float32),
    ),
    mesh=plsc.VectorSubcoreMesh(core_axis_name="c", subcore_axis_name="s"),
    scratch_types=[
        pltpu.VMEM((2 * NCH, CHUNK), jnp.int32),
        pltpu.VMEM((BPW, FACTOR), jnp.float32),
        pltpu.VMEM((BPW, FACTOR), jnp.float32),
        pltpu.SemaphoreType.DMA,
    ],
    compiler_params=pltpu.CompilerParams(use_tc_tiling_on_sc=False),
)(_sc_gather_body)


TC_BLK = 512


def _tc_affine_body(u_ref, i_ref, w_ref, b_ref, o_ref):
  x = u_ref[...] * i_ref[...]
  s = jnp.sum(x * w_ref[...], axis=1) + b_ref[0, 0]
  o_ref[...] = jax.nn.sigmoid(s)


def _tc_affine(u_rows, i_rows, affine_w, affine_b):
  grid = (BATCH // TC_BLK,)
  return pl.pallas_call(
      _tc_affine_body,
      grid=grid,
      in_specs=[
          pl.BlockSpec((TC_BLK, FACTOR), lambda i: (i, 0)),
          pl.BlockSpec((TC_BLK, FACTOR), lambda i: (i, 0)),
          pl.BlockSpec((1, FACTOR), lambda i: (0, 0)),
          pl.BlockSpec(memory_space=pltpu.SMEM),
      ],
      out_specs=pl.BlockSpec((TC_BLK,), lambda i: (i,)),
      out_shape=jax.ShapeDtypeStruct((BATCH,), jnp.float32),
  )(u_rows, i_rows, affine_w, affine_b.reshape(1, 1))


def kernel(user_indices, item_indices, user_table, item_table,
           affine_w, affine_b):
  uidx = user_indices.astype(jnp.int32)
  iidx = item_indices.astype(jnp.int32)
  u_rows, i_rows = _sc_gather(uidx, iidx, user_table, item_table)
  return _tc_affine(u_rows, i_rows, affine_w, affine_b)
